# baseline (speedup 1.0000x reference)
import jax
import jax.numpy as jnp
from jax.experimental import pallas as pl


def _edge_mlp_body(src_ref, dst_ref, ea_ref, w1a_ref, w1b_ref, w1c_ref,
                   b1_ref, w2_ref, b2_ref, out_ref):
    h = (src_ref[...] @ w1a_ref[...]
         + dst_ref[...] @ w1b_ref[...]
         + ea_ref[...] @ w1c_ref[...]
         + b1_ref[...])
    h = jnp.maximum(h, 0.0)
    out_ref[...] = h @ w2_ref[...] + b2_ref[...]


def _node_mlp_body(x_ref, agg_ref, wn1a_ref, wn1b_ref, bn1_ref, wn2_ref,
                   bn2_ref, wga_ref, wgb_ref, bg_ref, gate_ref, xnew_ref):
    x = x_ref[...]
    agg = agg_ref[...]
    hn = jnp.maximum(x @ wn1a_ref[...] + agg @ wn1b_ref[...] + bn1_ref[...], 0.0)
    x_upd = hn @ wn2_ref[...] + bn2_ref[...]
    gate = jax.nn.sigmoid(x @ wga_ref[...] + agg @ wgb_ref[...] + bg_ref[...])
    gate_ref[...] = gate
    xnew_ref[...] = gate * x_upd


def kernel(x, edge_index, edge_attr, u, batch, W_e1, b_e1, W_e2, b_e2,
           W_n1, b_n1, W_n2, b_n2, W_g, b_g, W_u1, b_u1, W_u2, b_u2):
    N, D = x.shape
    E = edge_index.shape[1]
    H = W_e2.shape[1]
    B, G = u.shape

    src = jnp.take(x, edge_index[0], axis=0)
    dst = jnp.take(x, edge_index[1], axis=0)

    W1a, W1b, W1c = W_e1[:D], W_e1[D:2 * D], W_e1[2 * D:]
    BE = 2000
    grid_e = E // BE
    b1 = b_e1.reshape(1, H)
    b2 = b_e2.reshape(1, H)
    message = pl.pallas_call(
        _edge_mlp_body,
        grid=(grid_e,),
        in_specs=[
            pl.BlockSpec((BE, D), lambda i: (i, 0)),
            pl.BlockSpec((BE, D), lambda i: (i, 0)),
            pl.BlockSpec((BE, edge_attr.shape[1]), lambda i: (i, 0)),
            pl.BlockSpec((D, H), lambda i: (0, 0)),
            pl.BlockSpec((D, H), lambda i: (0, 0)),
            pl.BlockSpec((edge_attr.shape[1], H), lambda i: (0, 0)),
            pl.BlockSpec((1, H), lambda i: (0, 0)),
            pl.BlockSpec((H, H), lambda i: (0, 0)),
            pl.BlockSpec((1, H), lambda i: (0, 0)),
        ],
        out_specs=pl.BlockSpec((BE, H), lambda i: (i, 0)),
        out_shape=jax.ShapeDtypeStruct((E, H), jnp.float32),
    )(src, dst, edge_attr, W1a, W1b, W1c, b1, W_e2, b2)

    agg = jax.ops.segment_sum(message, edge_index[1], num_segments=N)

    Wn1a, Wn1b = W_n1[:D], W_n1[D:]
    Wga, Wgb = W_g[:D], W_g[D:]
    BN = 1000
    grid_n = N // BN
    gate, x_new = pl.pallas_call(
        _node_mlp_body,
        grid=(grid_n,),
        in_specs=[
            pl.BlockSpec((BN, D), lambda i: (i, 0)),
            pl.BlockSpec((BN, H), lambda i: (i, 0)),
            pl.BlockSpec((D, H), lambda i: (0, 0)),
            pl.BlockSpec((H, H), lambda i: (0, 0)),
            pl.BlockSpec((1, H), lambda i: (0, 0)),
            pl.BlockSpec((H, H), lambda i: (0, 0)),
            pl.BlockSpec((1, H), lambda i: (0, 0)),
            pl.BlockSpec((D, H), lambda i: (0, 0)),
            pl.BlockSpec((H, H), lambda i: (0, 0)),
            pl.BlockSpec((1, H), lambda i: (0, 0)),
        ],
        out_specs=[
            pl.BlockSpec((BN, H), lambda i: (i, 0)),
            pl.BlockSpec((BN, H), lambda i: (i, 0)),
        ],
        out_shape=[
            jax.ShapeDtypeStruct((N, H), jnp.float32),
            jax.ShapeDtypeStruct((N, H), jnp.float32),
        ],
    )(x, agg, Wn1a, Wn1b, b_n1.reshape(1, H), W_n2, b_n2.reshape(1, H),
      Wga, Wgb, b_g.reshape(1, H))

    pooled = jax.ops.segment_sum(x_new, batch, num_segments=B)
    counts = jax.ops.segment_sum(jnp.ones((N,), jnp.float32), batch,
                                 num_segments=B)
    pooled = pooled / jnp.maximum(counts, 1.0)[:, None]
    g_in = jnp.concatenate([pooled, u], axis=-1)
    hg = jax.nn.relu(g_in @ W_u1 + b_u1)
    u_new = hg @ W_u2 + b_u2
    return (u_new, gate)


# speedup vs baseline: 2.5069x; 2.5069x over previous
import functools

import jax
import jax.numpy as jnp
from jax import lax
from jax.experimental import pallas as pl
from jax.experimental.pallas import tpu as pltpu
from jax.experimental.pallas import tpu_sc as plsc

N = 10000
E = 320000
D = 128
ED = 16
H = 128
G = 128
B = 16

NC = 2
NS = 16
HH = H // NC
EPT = E // NS
CK = 80
NCHUNK = EPT // CK
ZROWS = 640
NPAD = NS * ZROWS


def _pq_body(x_ref, w1a_ref, w1b_ref, p_ref, q_ref):
    xb = x_ref[...]
    p_ref[...] = xb @ w1a_ref[...]
    q_ref[...] = xb @ w1b_ref[...]


def _r_body(ea_ref, w1c_ref, b1_ref, r_ref):
    r_ref[...] = ea_ref[...] @ w1c_ref[...] + b1_ref[...]


def _sc_body(p_hbm, q_hbm, r_hbm, src_hbm, dst_hbm,
             s_out, d_out,
             sidx, didx, bufp, bufq, bufr, bufo,
             acc, sem0, sem1):
    c = lax.axis_index("c")
    s = lax.axis_index("s")
    zero16 = jnp.zeros((16,), jnp.float32)
    zbase = s * ZROWS

    def fill_bufo(val):
        def zrow(r_i, carry):
            for k in range(HH // 16):
                bufo[r_i, pl.ds(k * 16, 16)] = zero16 + val
            return carry

        lax.fori_loop(0, CK, zrow, None)

    def zero_acc():
        for j in range(ZROWS // CK):
            pltpu.sync_copy(bufo, acc.at[pl.ds(zbase + j * CK, CK), :])

    def read_acc_to(out_ref):
        for j in range(ZROWS // CK):
            rows = pl.ds(zbase + j * CK, CK)
            pltpu.sync_copy(acc.at[rows, :], bufo)
            pltpu.sync_copy(bufo, out_ref.at[rows, :])

    fill_bufo(0.0)
    zero_acc()
    plsc.subcore_barrier()

    def chunk(t, carry):
        base = s * EPT + t * CK
        pltpu.sync_copy(src_hbm.at[pl.ds(base, CK)], sidx)
        pltpu.sync_copy(dst_hbm.at[pl.ds(base, CK)], didx)
        cp_p = pltpu.async_copy(p_hbm.at[sidx], bufp, sem0)
        cp_q = pltpu.async_copy(q_hbm.at[didx], bufq, sem1)
        pltpu.sync_copy(r_hbm.at[pl.ds(base, CK), :], bufr)
        cp_p.wait()
        cp_q.wait()

        def compute_half(col0):
            def row(r_i, rcarry):
                for k in range(HH // 16):
                    sl = pl.ds(col0 + k * 16, 16)
                    v = bufp[r_i, sl] + bufq[r_i, sl] + bufr[r_i, sl]
                    bufo[r_i, pl.ds(k * 16, 16)] = jnp.maximum(v, 0.0)
                return rcarry

            lax.fori_loop(0, CK, row, None)

        @pl.when(c == 0)
        def _():
            compute_half(0)

        @pl.when(c == 1)
        def _():
            compute_half(HH)

        pltpu.sync_copy(bufo, acc.at[didx], add=True)
        return carry

    lax.fori_loop(0, NCHUNK, chunk, None)
    plsc.subcore_barrier()
    read_acc_to(s_out.at[c])
    plsc.subcore_barrier()

    fill_bufo(0.0)
    zero_acc()
    plsc.subcore_barrier()
    fill_bufo(1.0)

    def dchunk(t, carry):
        base = s * EPT + c * (EPT // 2) + t * CK
        pltpu.sync_copy(dst_hbm.at[pl.ds(base, CK)], didx)
        pltpu.sync_copy(bufo, acc.at[didx], add=True)
        return carry

    lax.fori_loop(0, NCHUNK // 2, dchunk, None)
    plsc.subcore_barrier()
    read_acc_to(d_out.at[c])


def _node_body(s0_ref, s1_ref, d0_ref, d1_ref, x_ref, oh_ref,
               we2_ref, be2_ref, wn1a_ref, wn1b_ref, bn1_ref,
               wn2_ref, bn2_ref, wga_ref, wgb_ref, bg_ref,
               gate_ref, pc_ref):
    i = pl.program_id(0)
    ssum = jnp.concatenate([s0_ref[0], s1_ref[0]], axis=-1)
    deg = d0_ref[0][:, :1] + d1_ref[0][:, :1]
    agg = ssum @ we2_ref[...] + deg * be2_ref[...]
    xb = x_ref[...]
    hn = jnp.maximum(xb @ wn1a_ref[...] + agg @ wn1b_ref[...] + bn1_ref[...],
                     0.0)
    x_upd = hn @ wn2_ref[...] + bn2_ref[...]
    gate = jax.nn.sigmoid(xb @ wga_ref[...] + agg @ wgb_ref[...] + bg_ref[...])
    x_new = gate * x_upd
    gate_ref[...] = gate
    oh = oh_ref[...]
    pooled = lax.dot_general(oh, x_new, (((0,), (0,)), ((), ())))
    cnt = lax.dot_general(oh, jnp.ones_like(x_new), (((0,), (0,)), ((), ())))
    blk = jnp.concatenate([pooled, cnt], axis=0)

    @pl.when(i == 0)
    def _():
        pc_ref[...] = blk

    @pl.when(i > 0)
    def _():
        pc_ref[...] += blk


def _global_body(pc_ref, u_ref, wua_ref, wub_ref, bu1_ref, wu2_ref, bu2_ref,
                 out_ref):
    pooled = pc_ref[:B, :] / jnp.maximum(pc_ref[B:, :], 1.0)
    hg = jnp.maximum(pooled @ wua_ref[...] + u_ref[...] @ wub_ref[...]
                     + bu1_ref[...], 0.0)
    out_ref[...] = hg @ wu2_ref[...] + bu2_ref[...]


def kernel(x, edge_index, edge_attr, u, batch, W_e1, b_e1, W_e2, b_e2,
           W_n1, b_n1, W_n2, b_n2, W_g, b_g, W_u1, b_u1, W_u2, b_u2):
    W1a, W1b, W1c = W_e1[:D], W_e1[D:2 * D], W_e1[2 * D:]

    BN = 2000
    P, Q = pl.pallas_call(
        _pq_body,
        grid=(N // BN,),
        in_specs=[
            pl.BlockSpec((BN, D), lambda i: (i, 0)),
            pl.BlockSpec((D, H), lambda i: (0, 0)),
            pl.BlockSpec((D, H), lambda i: (0, 0)),
        ],
        out_specs=[
            pl.BlockSpec((BN, H), lambda i: (i, 0)),
            pl.BlockSpec((BN, H), lambda i: (i, 0)),
        ],
        out_shape=[
            jax.ShapeDtypeStruct((N, H), jnp.float32),
            jax.ShapeDtypeStruct((N, H), jnp.float32),
        ],
    )(x, W1a, W1b)

    BE = 4000
    R = pl.pallas_call(
        _r_body,
        grid=(E // BE,),
        in_specs=[
            pl.BlockSpec((BE, ED), lambda i: (i, 0)),
            pl.BlockSpec((ED, H), lambda i: (0, 0)),
            pl.BlockSpec((1, H), lambda i: (0, 0)),
        ],
        out_specs=pl.BlockSpec((BE, H), lambda i: (i, 0)),
        out_shape=jax.ShapeDtypeStruct((E, H), jnp.float32),
    )(edge_attr, W1c, b_e1.reshape(1, H))

    sc = functools.partial(
        pl.kernel,
        mesh=plsc.VectorSubcoreMesh(core_axis_name="c", subcore_axis_name="s"),
        out_type=[
            pltpu.HBM((NC, NPAD, HH), jnp.float32),
            pltpu.HBM((NC, NPAD, HH), jnp.float32),
        ],
        scratch_types=[
            pltpu.VMEM((CK,), jnp.int32),
            pltpu.VMEM((CK,), jnp.int32),
            pltpu.VMEM((CK, H), jnp.float32),
            pltpu.VMEM((CK, H), jnp.float32),
            pltpu.VMEM((CK, H), jnp.float32),
            pltpu.VMEM((CK, HH), jnp.float32),
            pltpu.VMEM_SHARED((NPAD, HH), jnp.float32),
            pltpu.SemaphoreType.DMA,
            pltpu.SemaphoreType.DMA,
        ],
    )(_sc_body)
    S_out, Deg_out = sc(P, Q, R, edge_index[0], edge_index[1])

    oh = (batch[:, None] == jnp.arange(B, dtype=jnp.int32)[None, :]
          ).astype(jnp.float32)

    BNN = 1000
    gate, pc = pl.pallas_call(
        _node_body,
        grid=(N // BNN,),
        in_specs=[
            pl.BlockSpec((1, BNN, HH), lambda i: (0, i, 0)),
            pl.BlockSpec((1, BNN, HH), lambda i: (1, i, 0)),
            pl.BlockSpec((1, BNN, HH), lambda i: (0, i, 0)),
            pl.BlockSpec((1, BNN, HH), lambda i: (1, i, 0)),
            pl.BlockSpec((BNN, D), lambda i: (i, 0)),
            pl.BlockSpec((BNN, B), lambda i: (i, 0)),
            pl.BlockSpec((H, H), lambda i: (0, 0)),
            pl.BlockSpec((1, H), lambda i: (0, 0)),
            pl.BlockSpec((D, H), lambda i: (0, 0)),
            pl.BlockSpec((H, H), lambda i: (0, 0)),
            pl.BlockSpec((1, H), lambda i: (0, 0)),
            pl.BlockSpec((H, H), lambda i: (0, 0)),
            pl.BlockSpec((1, H), lambda i: (0, 0)),
            pl.BlockSpec((D, H), lambda i: (0, 0)),
            pl.BlockSpec((H, H), lambda i: (0, 0)),
            pl.BlockSpec((1, H), lambda i: (0, 0)),
        ],
        out_specs=[
            pl.BlockSpec((BNN, H), lambda i: (i, 0)),
            pl.BlockSpec((2 * B, H), lambda i: (0, 0)),
        ],
        out_shape=[
            jax.ShapeDtypeStruct((N, H), jnp.float32),
            jax.ShapeDtypeStruct((2 * B, H), jnp.float32),
        ],
    )(S_out, S_out, Deg_out, Deg_out, x, oh,
      W_e2, b_e2.reshape(1, H), W_n1[:D], W_n1[D:], b_n1.reshape(1, H),
      W_n2, b_n2.reshape(1, H), W_g[:D], W_g[D:], b_g.reshape(1, H))

    u_new = pl.pallas_call(
        _global_body,
        in_specs=[
            pl.BlockSpec((2 * B, H), lambda: (0, 0)),
            pl.BlockSpec((B, G), lambda: (0, 0)),
            pl.BlockSpec((H, H), lambda: (0, 0)),
            pl.BlockSpec((G, H), lambda: (0, 0)),
            pl.BlockSpec((1, H), lambda: (0, 0)),
            pl.BlockSpec((H, G), lambda: (0, 0)),
            pl.BlockSpec((1, G), lambda: (0, 0)),
        ],
        out_specs=pl.BlockSpec((B, G), lambda: (0, 0)),
        out_shape=jax.ShapeDtypeStruct((B, G), jnp.float32),
    )(pc, u, W_u1[:H], W_u1[H:], b_u1.reshape(1, H), W_u2,
      b_u2.reshape(1, G))

    return (u_new, gate)
